# baseline (speedup 1.0000x reference)
import jax
import jax.numpy as jnp
from jax.experimental import pallas as pl

_BN = 2048
_CHUNK = 4096


def _vq_block_kernel(flat_ref, cbt_ref, out_ref):
    flat = flat_ref[...]
    cbt = cbt_ref[...]
    n, k = flat.shape[0], cbt.shape[1]

    z_bf = (flat + flat).astype(jnp.bfloat16)
    cb_bf = cbt.astype(jnp.bfloat16)
    dims = (((1,), (0,)), ((), ()))
    mm2 = jax.lax.dot_general(z_bf, cb_bf, dims,
                              preferred_element_type=jnp.float32)

    i_sqr = jnp.sum(flat * flat, axis=1, keepdims=True)
    c_sqr = jnp.sum(cbt * cbt, axis=0, keepdims=True)
    dist = (i_sqr + c_sqr) - mm2

    iota = jax.lax.broadcasted_iota(jnp.int32, (n, _CHUNK), 1)
    accv = jnp.full((n, 1), jnp.inf, dtype=jnp.float32)
    acci = jnp.zeros((n, 1), dtype=jnp.int32)
    for c in range(k // _CHUNK):
        sub = dist[:, c * _CHUNK:(c + 1) * _CHUNK]
        mv = jnp.min(sub, axis=1, keepdims=True)
        mi = jnp.min(jnp.where(sub == mv, iota, _CHUNK),
                     axis=1, keepdims=True) + c * _CHUNK
        keep = accv <= mv
        acci = jnp.where(keep, acci, mi)
        accv = jnp.where(keep, accv,
                         mv.astype(jnp.bfloat16).astype(jnp.float32))
    out_ref[...] = acci.reshape(1, 1, n)


def kernel(z_e_x, codebook):
    B, T, D = z_e_x.shape
    K = codebook.shape[0]
    N = B * T
    flat = z_e_x.reshape(N, D)
    cbt = codebook.T
    nblk = N // _BN
    out = pl.pallas_call(
        _vq_block_kernel,
        grid=(nblk,),
        in_specs=[
            pl.BlockSpec((_BN, D), lambda i: (i, 0)),
            pl.BlockSpec((D, K), lambda i: (0, 0)),
        ],
        out_specs=pl.BlockSpec((1, 1, _BN), lambda i: (i, 0, 0)),
        out_shape=jax.ShapeDtypeStruct((nblk, 1, _BN), jnp.int32),
    )(flat, cbt)
    return out.reshape(B, T)


# speedup vs baseline: 1.1060x; 1.1060x over previous
import jax
import jax.numpy as jnp
from jax.experimental import pallas as pl

_BN = 2048
_CHUNK = 4096


def _vq_block_kernel(flat_ref, cbt_ref, out_ref):
    flat = flat_ref[...]
    cbt = cbt_ref[...]
    n, k = flat.shape[0], cbt.shape[1]

    z_bf = flat.astype(jnp.bfloat16)
    cb_bf = cbt.astype(jnp.bfloat16)
    dims = (((1,), (0,)), ((), ()))
    mm = jax.lax.dot_general(z_bf, cb_bf, dims,
                             preferred_element_type=jnp.float32)

    i_sqr = jnp.sum(flat * flat, axis=1, keepdims=True)
    c_sqr = jnp.sum(cbt * cbt, axis=0, keepdims=True)
    dist = (i_sqr + c_sqr) - 2.0 * mm

    iota = jax.lax.broadcasted_iota(jnp.int32, (n, _CHUNK), 1)
    accv = jnp.full((n, 1), jnp.inf, dtype=jnp.float32)
    acci = jnp.zeros((n, 1), dtype=jnp.int32)
    for c in range(k // _CHUNK):
        sub = dist[:, c * _CHUNK:(c + 1) * _CHUNK]
        mv = jnp.min(sub, axis=1, keepdims=True)
        mi = jnp.min(jnp.where(sub == mv, iota, _CHUNK),
                     axis=1, keepdims=True) + c * _CHUNK
        bits = jax.lax.bitcast_convert_type(mv, jnp.uint32)
        bits = (bits + jnp.uint32(0x7FFF)) & jnp.uint32(0xFFFF0000)
        mv_bf = jax.lax.bitcast_convert_type(bits, jnp.float32)
        keep = accv < mv
        acci = jnp.where(keep, acci, mi)
        accv = jnp.where(keep, accv, mv_bf)
    out_ref[...] = acci.reshape(1, 1, n)


def kernel(z_e_x, codebook):
    B, T, D = z_e_x.shape
    K = codebook.shape[0]
    N = B * T
    flat = z_e_x.reshape(N, D)
    cbt = codebook.T
    nblk = N // _BN
    out = pl.pallas_call(
        _vq_block_kernel,
        grid=(nblk,),
        in_specs=[
            pl.BlockSpec((_BN, D), lambda i: (i, 0)),
            pl.BlockSpec((D, K), lambda i: (0, 0)),
        ],
        out_specs=pl.BlockSpec((1, 1, _BN), lambda i: (i, 0, 0)),
        out_shape=jax.ShapeDtypeStruct((nblk, 1, _BN), jnp.int32),
    )(flat, cbt)
    return out.reshape(B, T)
